# Initial kernel scaffold; baseline (speedup 1.0000x reference)
import jax
import jax.numpy as jnp
from jax.experimental import pallas as pl


def kernel(rpn_cls_scores, rpn_bbox_preds, img_size):
    raise NotImplementedError("write your pallas kernel here")


# speedup vs baseline: 23.7177x; 23.7177x over previous
import numpy as np
import jax
import jax.numpy as jnp
from jax.experimental import pallas as pl
from jax.experimental.pallas import tpu as pltpu

_N_ANCHOR = 9
_FH = 64
_FW = 64
_FEAT_STRIDE = 16
_N = _FH * _FW * _N_ANCHOR
_LANES = 128
_ROWS = _N // _LANES
_PRE = 12000
_POST = 2000
_OROWS = 16
_THRESH = 0.7
_MIN_SIZE = 16.0


def _np_base_anchors(base_size=16, ratios=(0.5, 1.0, 2.0), scales=(8, 16, 32)):
    ab = np.zeros((len(ratios) * len(scales), 4), dtype=np.float32)
    px = base_size / 2.0
    py = base_size / 2.0
    for i, r in enumerate(ratios):
        for j, s in enumerate(scales):
            h = base_size * s * np.sqrt(r)
            w = base_size * s * np.sqrt(1.0 / r)
            k = i * len(scales) + j
            ab[k, 0] = px - w / 2.0
            ab[k, 1] = py - h / 2.0
            ab[k, 2] = px + w / 2.0
            ab[k, 3] = py + h / 2.0
    return ab


def _np_all_anchors():
    base = _np_base_anchors()
    shift_x = np.arange(_FW) * _FEAT_STRIDE
    shift_y = np.arange(_FH) * _FEAT_STRIDE
    sx, sy = np.meshgrid(shift_x, shift_y)
    shifts = np.stack([sx.ravel(), sy.ravel(), sx.ravel(), sy.ravel()], axis=1).astype(np.float32)
    return (shifts[:, None, :] + base[None, :, :]).reshape(-1, 4)


_ANC = _np_all_anchors()
_AW = (_ANC[:, 2] - _ANC[:, 0] + 1.0).reshape(_ROWS, _LANES)
_AH = (_ANC[:, 3] - _ANC[:, 1] + 1.0).reshape(_ROWS, _LANES)
_ACX = (_ANC[:, 0].reshape(_ROWS, _LANES) + 0.5 * _AW)
_ACY = (_ANC[:, 1].reshape(_ROWS, _LANES) + 0.5 * _AH)


def _nms_kernel(bounds_ref, z0_ref, z1_ref, dx_ref, dy_ref, dw_ref, dh_ref,
                aw_ref, ah_ref, acx_ref, acy_ref,
                ox1_ref, oy1_ref, ox2_ref, oy2_ref,
                s_ref, px1_ref, py1_ref, px2_ref, py2_ref, par_ref):
    ninf = jnp.float32(-jnp.inf)

    z0 = z0_ref[...]
    z1 = z1_ref[...]
    zm = jnp.maximum(z0, z1)
    e0 = jnp.exp(z0 - zm)
    e1 = jnp.exp(z1 - zm)
    scores = e1 / (e0 + e1)

    aw = aw_ref[...]
    ah = ah_ref[...]
    pcx = dx_ref[...] * aw + acx_ref[...]
    pcy = dy_ref[...] * ah + acy_ref[...]
    pw = jnp.exp(dw_ref[...]) * aw
    ph = jnp.exp(dh_ref[...]) * ah
    hi_h = bounds_ref[0, 0]
    hi_w = bounds_ref[0, 1]
    x1 = jnp.minimum(jnp.maximum(pcx - 0.5 * pw, 0.0), hi_w)
    y1 = jnp.minimum(jnp.maximum(pcy - 0.5 * ph, 0.0), hi_h)
    x2 = jnp.minimum(jnp.maximum(pcx + 0.5 * pw, 0.0), hi_w)
    y2 = jnp.minimum(jnp.maximum(pcy + 0.5 * ph, 0.0), hi_h)
    ws = x2 - x1 + 1.0
    hs = y2 - y1 + 1.0
    valid = (ws >= _MIN_SIZE) & (hs >= _MIN_SIZE)
    ms = jnp.where(valid, scores, ninf)

    kk = jax.lax.bitcast_convert_type(ms, jnp.int32)
    one = jnp.int32(1)

    def _sel_bit(j, t):
        cand = t | jax.lax.shift_left(one, 30 - j)
        cnt = jnp.sum(jnp.where(kk >= cand, one, jnp.int32(0)))
        return jnp.where(cnt >= _PRE, cand, t)

    t = jax.lax.fori_loop(0, 31, _sel_bit, jnp.int32(0))

    li = (jax.lax.broadcasted_iota(jnp.int32, (_ROWS, _LANES), 0) * _LANES
          + jax.lax.broadcasted_iota(jnp.int32, (_ROWS, _LANES), 1))
    cls_mask = (kk == t)
    c_gt = jnp.sum(jnp.where(kk > t, one, jnp.int32(0)))
    needed = jnp.int32(_PRE) - c_gt

    def _sel_idx(j, cc):
        cand = cc | jax.lax.shift_left(one, 15 - j)
        cnt = jnp.sum(jnp.where(cls_mask & (li < cand), one, jnp.int32(0)))
        return jnp.where(cnt >= needed, cc, cand)

    cutoff = jax.lax.fori_loop(0, 16, _sel_idx, jnp.int32(0))
    keep = (kk > t) | (cls_mask & (li <= cutoff))

    s_ref[...] = jnp.where(keep, ms, ninf)
    px1_ref[...] = x1
    py1_ref[...] = y1
    px2_ref[...] = x2
    py2_ref[...] = y2
    par_ref[...] = (x2 - x1 + 1.0) * (y2 - y1 + 1.0)

    zo = jnp.zeros((_OROWS, _LANES), jnp.float32)
    ox1_ref[...] = zo
    oy1_ref[...] = zo
    ox2_ref[...] = zo
    oy2_ref[...] = zo

    li_o = (jax.lax.broadcasted_iota(jnp.int32, (_OROWS, _LANES), 0) * _LANES
            + jax.lax.broadcasted_iota(jnp.int32, (_OROWS, _LANES), 1))
    lane = jax.lax.broadcasted_iota(jnp.int32, (1, _LANES), 1)

    def _step(i, carry):
        s = s_ref[...]
        m = jnp.max(s)
        idx = jnp.min(jnp.where(s == m, li, jnp.int32(_N)))
        r = idx // _LANES
        c = idx - r * _LANES
        sel = lane == c
        bx1 = jnp.max(jnp.where(sel, px1_ref[pl.ds(r, 1), :], ninf))
        by1 = jnp.max(jnp.where(sel, py1_ref[pl.ds(r, 1), :], ninf))
        bx2 = jnp.max(jnp.where(sel, px2_ref[pl.ds(r, 1), :], ninf))
        by2 = jnp.max(jnp.where(sel, py2_ref[pl.ds(r, 1), :], ninf))
        ba = jnp.max(jnp.where(sel, par_ref[pl.ds(r, 1), :], ninf))

        xx1 = jnp.maximum(bx1, px1_ref[...])
        yy1 = jnp.maximum(by1, py1_ref[...])
        xx2 = jnp.minimum(bx2, px2_ref[...])
        yy2 = jnp.minimum(by2, py2_ref[...])
        w = jnp.maximum(0.0, xx2 - xx1 + 1.0)
        h = jnp.maximum(0.0, yy2 - yy1 + 1.0)
        inter = w * h
        iou = inter / (ba + par_ref[...] - inter)
        s_ref[...] = jnp.where((iou > _THRESH) | (li == idx), ninf, s)

        vf = jnp.where(m > ninf, jnp.float32(1.0), jnp.float32(0.0))
        om = li_o == i
        ox1_ref[...] = jnp.where(om, bx1 * vf, ox1_ref[...])
        oy1_ref[...] = jnp.where(om, by1 * vf, oy1_ref[...])
        ox2_ref[...] = jnp.where(om, bx2 * vf, ox2_ref[...])
        oy2_ref[...] = jnp.where(om, by2 * vf, oy2_ref[...])
        return carry

    jax.lax.fori_loop(0, _POST, _step, jnp.int32(0))


def _run(bounds, z0, z1, dx, dy, dw, dh, aw, ah, acx, acy):
    vspec = pl.BlockSpec(memory_space=pltpu.VMEM)
    return pl.pallas_call(
        _nms_kernel,
        out_shape=[jax.ShapeDtypeStruct((_OROWS, _LANES), jnp.float32)] * 4,
        in_specs=[pl.BlockSpec(memory_space=pltpu.SMEM)] + [vspec] * 10,
        out_specs=[vspec] * 4,
        scratch_shapes=[pltpu.VMEM((_ROWS, _LANES), jnp.float32)] * 6,
    )(bounds, z0, z1, dx, dy, dw, dh, aw, ah, acx, acy)


def kernel(rpn_cls_scores, rpn_bbox_preds, img_size):
    cls = rpn_cls_scores.reshape(_N_ANCHOR, 2, _FH, _FW)
    z0 = jnp.transpose(cls[:, 0], (1, 2, 0)).reshape(_ROWS, _LANES)
    z1 = jnp.transpose(cls[:, 1], (1, 2, 0)).reshape(_ROWS, _LANES)
    bb = rpn_bbox_preds.reshape(_N_ANCHOR, 4, _FH, _FW)
    dx = jnp.transpose(bb[:, 0], (1, 2, 0)).reshape(_ROWS, _LANES)
    dy = jnp.transpose(bb[:, 1], (1, 2, 0)).reshape(_ROWS, _LANES)
    dw = jnp.transpose(bb[:, 2], (1, 2, 0)).reshape(_ROWS, _LANES)
    dh = jnp.transpose(bb[:, 3], (1, 2, 0)).reshape(_ROWS, _LANES)
    fimg = img_size.astype(jnp.float32)
    bounds = (fimg - 1.0).reshape(1, 2)
    aw = jnp.asarray(_AW)
    ah = jnp.asarray(_AH)
    acx = jnp.asarray(_ACX)
    acy = jnp.asarray(_ACY)
    ox1, oy1, ox2, oy2 = _run(bounds, z0, z1, dx, dy, dw, dh, aw, ah, acx, acy)
    bx1 = ox1.reshape(-1)[:_POST]
    by1 = oy1.reshape(-1)[:_POST]
    bx2 = ox2.reshape(-1)[:_POST]
    by2 = oy2.reshape(-1)[:_POST]
    batch = jnp.zeros((_POST, 1), jnp.float32)
    return jnp.concatenate([batch, jnp.stack([bx1, by1, bx2, by2], axis=1)], axis=1)
